# baseline (speedup 1.0000x reference)
import functools

import jax
import jax.numpy as jnp
from jax import lax
from jax.experimental import pallas as pl
from jax.experimental.pallas import tpu as pltpu
from jax.experimental.pallas import tpu_sc as plsc

NC, NS = 2, 16
NW = NC * NS
CHUNK = 16
NSETS = 4
LANES = 16


def _sc_gather_sum(row_idx, col_idx, row_table, col_table):
  n = row_idx.shape[0]
  d = row_table.shape[1]
  per_w = n // NW
  steps = per_w // CHUNK
  mesh = plsc.VectorSubcoreMesh(core_axis_name="c", subcore_axis_name="s")

  @functools.partial(
      pl.kernel,
      out_type=jax.ShapeDtypeStruct((n, d), row_table.dtype),
      mesh=mesh,
      scratch_types=(
          [pltpu.VMEM((per_w,), jnp.int32)] * 2
          + [pltpu.VMEM((CHUNK, d), row_table.dtype)] * (2 * NSETS)
          + [pltpu.SemaphoreType.DMA] * (2 * NSETS)
      ),
  )
  def k(ri_hbm, ci_hbm, rt_hbm, ct_hbm, g_hbm, ir_v, ic_v, *rest):
    bufs = rest[:2 * NSETS]
    sems = rest[2 * NSETS:]
    wid = lax.axis_index("s") * NC + lax.axis_index("c")
    base = wid * per_w
    pltpu.sync_copy(ri_hbm.at[pl.ds(base, per_w)], ir_v)
    pltpu.sync_copy(ci_hbm.at[pl.ds(base, per_w)], ic_v)
    sets = tuple(
        (bufs[2 * j], bufs[2 * j + 1], sems[2 * j], sems[2 * j + 1])
        for j in range(NSETS))
    handles = [None] * (2 * steps)

    def start(k_):
      rbuf, cbuf, rs, cs = sets[k_ % NSETS]
      handles[2 * k_] = pltpu.async_copy(
          rt_hbm.at[ir_v.at[pl.ds(k_ * CHUNK, CHUNK)]], rbuf, rs)
      handles[2 * k_ + 1] = pltpu.async_copy(
          ct_hbm.at[ic_v.at[pl.ds(k_ * CHUNK, CHUNK)]], cbuf, cs)

    def finish(k_):
      rbuf, cbuf, _, _ = sets[k_ % NSETS]
      handles[2 * k_].wait()
      handles[2 * k_ + 1].wait()

      @pl.loop(0, CHUNK)
      def _(r):
        for c in range(d // LANES):
          sl = (r, pl.ds(c * LANES, LANES))
          rbuf[sl] = rbuf[sl] + cbuf[sl]

      pltpu.sync_copy(rbuf, g_hbm.at[pl.ds(base + k_ * CHUNK, CHUNK)])

    for k_ in range(NSETS - 1):
      start(k_)
    for k_ in range(NSETS - 1, steps):
      start(k_)
      finish(k_ - (NSETS - 1))
    for k_ in range(steps - (NSETS - 1), steps):
      finish(k_)

  return k(row_idx, col_idx, row_table, col_table)


def _tc_body(fpt_ref, w_ref, b_ref, g_ref, out_ref):
  w = w_ref[...].astype(jnp.bfloat16)
  x = fpt_ref[...].astype(jnp.bfloat16)
  acc = lax.dot_general(x, w, (((0,), (0,)), ((), ())),
                        preferred_element_type=jnp.float32)
  out_ref[...] = acc + b_ref[...] + g_ref[...]


def _tc_project_add(fpt3, w_pad, b2, g3, block_cols=512):
  pw, bsz, s = fpt3.shape
  h = w_pad.shape[1]
  grid = (s // block_cols,)
  return pl.pallas_call(
      _tc_body,
      grid=grid,
      in_specs=[
          pl.BlockSpec((pw, bsz, block_cols), lambda i: (0, 0, i)),
          pl.BlockSpec((pw, h), lambda i: (0, 0)),
          pl.BlockSpec((1, h), lambda i: (0, 0)),
          pl.BlockSpec((bsz, block_cols, h), lambda i: (0, i, 0)),
      ],
      out_specs=pl.BlockSpec((bsz, block_cols, h), lambda i: (0, i, 0)),
      out_shape=jax.ShapeDtypeStruct((bsz, s, h), jnp.float32),
  )(fpt3, w_pad, b2, g3)


def kernel(flattened_patches, W, b, row_table, col_table):
  bsz, s, pw = flattened_patches.shape
  h = W.shape[1]
  n = bsz * s
  fpt3 = flattened_patches.transpose(2, 0, 1)
  row_idx = fpt3[0].reshape(n).astype(jnp.int32)
  col_idx = fpt3[1].reshape(n).astype(jnp.int32)
  g = _sc_gather_sum(row_idx, col_idx, row_table, col_table)
  w_pad = jnp.concatenate([jnp.zeros((2, h), W.dtype), W], axis=0)
  out = _tc_project_add(fpt3, w_pad, b.reshape(1, h), g.reshape(bsz, s, h))
  return out


# speedup vs baseline: 1.0489x; 1.0489x over previous
import functools

import jax
import jax.numpy as jnp
from jax import lax
from jax.experimental import pallas as pl
from jax.experimental.pallas import tpu as pltpu
from jax.experimental.pallas import tpu_sc as plsc

NC, NS = 2, 16
NW = NC * NS
CHUNK = 32
NSETS = 2
LANES = 16


def _sc_gather_sum(row_idx, col_idx, row_table, col_table):
  n = row_idx.shape[0]
  d = row_table.shape[1]
  per_w = n // NW
  steps = per_w // CHUNK
  mesh = plsc.VectorSubcoreMesh(core_axis_name="c", subcore_axis_name="s")

  @functools.partial(
      pl.kernel,
      out_type=jax.ShapeDtypeStruct((n, d), row_table.dtype),
      mesh=mesh,
      scratch_types=(
          [pltpu.VMEM((per_w,), jnp.int32)] * 2
          + [pltpu.VMEM((CHUNK, d), row_table.dtype)] * (2 * NSETS)
          + [pltpu.SemaphoreType.DMA] * (2 * NSETS)
      ),
  )
  def k(ri_hbm, ci_hbm, rt_hbm, ct_hbm, g_hbm, ir_v, ic_v, *rest):
    bufs = rest[:2 * NSETS]
    sems = rest[2 * NSETS:]
    wid = lax.axis_index("s") * NC + lax.axis_index("c")
    base = wid * per_w
    pltpu.sync_copy(ri_hbm.at[pl.ds(base, per_w)], ir_v)
    pltpu.sync_copy(ci_hbm.at[pl.ds(base, per_w)], ic_v)
    sets = tuple(
        (bufs[2 * j], bufs[2 * j + 1], sems[2 * j], sems[2 * j + 1])
        for j in range(NSETS))
    handles = [None] * (2 * steps)

    def start(k_):
      rbuf, cbuf, rs, cs = sets[k_ % NSETS]
      handles[2 * k_] = pltpu.async_copy(
          rt_hbm.at[ir_v.at[pl.ds(k_ * CHUNK, CHUNK)]], rbuf, rs)
      handles[2 * k_ + 1] = pltpu.async_copy(
          ct_hbm.at[ic_v.at[pl.ds(k_ * CHUNK, CHUNK)]], cbuf, cs)

    def finish(k_):
      rbuf, cbuf, _, _ = sets[k_ % NSETS]
      handles[2 * k_].wait()
      handles[2 * k_ + 1].wait()

      @pl.loop(0, CHUNK)
      def _(r):
        for c in range(d // LANES):
          sl = (r, pl.ds(c * LANES, LANES))
          rbuf[sl] = rbuf[sl] + cbuf[sl]

      pltpu.sync_copy(rbuf, g_hbm.at[pl.ds(base + k_ * CHUNK, CHUNK)])

    for k_ in range(NSETS - 1):
      start(k_)
    for k_ in range(NSETS - 1, steps):
      start(k_)
      finish(k_ - (NSETS - 1))
    for k_ in range(steps - (NSETS - 1), steps):
      finish(k_)

  return k(row_idx, col_idx, row_table, col_table)


def _tc_body(fpt_ref, w_ref, b_ref, g_ref, out_ref):
  w = w_ref[...].astype(jnp.bfloat16)
  x = fpt_ref[...].astype(jnp.bfloat16)
  acc = lax.dot_general(x, w, (((0,), (0,)), ((), ())),
                        preferred_element_type=jnp.float32)
  out_ref[...] = acc + b_ref[...] + g_ref[...]


def _tc_project_add(fpt3, w_pad, b2, g3, block_cols=512):
  pw, bsz, s = fpt3.shape
  h = w_pad.shape[1]
  grid = (s // block_cols,)
  return pl.pallas_call(
      _tc_body,
      grid=grid,
      in_specs=[
          pl.BlockSpec((pw, bsz, block_cols), lambda i: (0, 0, i)),
          pl.BlockSpec((pw, h), lambda i: (0, 0)),
          pl.BlockSpec((1, h), lambda i: (0, 0)),
          pl.BlockSpec((bsz, block_cols, h), lambda i: (0, i, 0)),
      ],
      out_specs=pl.BlockSpec((bsz, block_cols, h), lambda i: (0, i, 0)),
      out_shape=jax.ShapeDtypeStruct((bsz, s, h), jnp.float32),
  )(fpt3, w_pad, b2, g3)


def kernel(flattened_patches, W, b, row_table, col_table):
  bsz, s, pw = flattened_patches.shape
  h = W.shape[1]
  n = bsz * s
  fpt3 = flattened_patches.transpose(2, 0, 1)
  row_idx = fpt3[0].reshape(n).astype(jnp.int32)
  col_idx = fpt3[1].reshape(n).astype(jnp.int32)
  g = _sc_gather_sum(row_idx, col_idx, row_table, col_table)
  w_pad = jnp.concatenate([jnp.zeros((2, h), W.dtype), W], axis=0)
  out = _tc_project_add(fpt3, w_pad, b.reshape(1, h), g.reshape(bsz, s, h))
  return out
